# baseline (speedup 1.0000x reference)
import jax
import jax.numpy as jnp
from jax.experimental import pallas as pl

_B, _N, _M = 16, 2048, 2048
_NT = 512
_K = 8


def _bf16_parts(x, n):
    parts = []
    for _ in range(n):
        hi = x.astype(jnp.bfloat16).astype(jnp.float32)
        parts.append(hi)
        x = x - hi
    return parts


def _chamfer_body(s_ref, t_ref, out_ref):
    t = t_ref[0]
    col_min = jnp.full((1, _M), jnp.inf, dtype=jnp.float32)
    row_total = jnp.float32(0.0)
    for i in range(_N // _NT):
        s = s_ref[0, i * _NT:(i + 1) * _NT, :]
        d2 = jax.lax.dot_general(
            s, t, (((1,), (0,)), ((), ())),
            precision=jax.lax.Precision.DEFAULT,
            preferred_element_type=jnp.float32)
        row_total = row_total + jnp.sum(jnp.min(d2, axis=1))
        col_min = jnp.minimum(col_min, jnp.min(d2, axis=0, keepdims=True))
    res = row_total / _N + jnp.sum(col_min) / _M
    out_ref[0] = jnp.reshape(res, (1, 1))


def kernel(src, tgt):
    x2 = jnp.sum(src * src, axis=-1)
    y2 = jnp.sum(tgt * tgt, axis=-1)
    x2h, x2m, x2l = _bf16_parts(x2, 3)
    y2h, y2l = _bf16_parts(y2, 2)
    ones_n = jnp.ones_like(x2)
    ones_m = jnp.ones_like(y2)
    s_aug = jnp.stack(
        [-2.0 * src[..., 0], -2.0 * src[..., 1], -2.0 * src[..., 2],
         x2h, x2m, x2l, ones_n, ones_n], axis=-1)
    t_aug = jnp.stack(
        [tgt[..., 0], tgt[..., 1], tgt[..., 2],
         ones_m, ones_m, ones_m, y2h, y2l], axis=1)
    per_batch = pl.pallas_call(
        _chamfer_body,
        grid=(_B,),
        in_specs=[
            pl.BlockSpec((1, _N, _K), lambda b: (b, 0, 0)),
            pl.BlockSpec((1, _K, _M), lambda b: (b, 0, 0)),
        ],
        out_specs=pl.BlockSpec((1, 1, 1), lambda b: (b, 0, 0)),
        out_shape=jax.ShapeDtypeStruct((_B, 1, 1), jnp.float32),
    )(s_aug, t_aug)
    return jnp.sum(per_batch)


# speedup vs baseline: 2.7383x; 2.7383x over previous
import jax
import jax.numpy as jnp
from jax.experimental import pallas as pl

_B, _N, _M = 16, 2048, 2048
_NT = 512


def _parts(x, n):
    out = []
    for _ in range(n):
        hi = x.astype(jnp.bfloat16).astype(jnp.float32)
        out.append(hi)
        x = x - hi
    return out


def _chamfer_body(src_ref, tgt_ref, out_ref):
    t3 = tgt_ref[0]
    y2 = jnp.sum(t3 * t3, axis=1, keepdims=True)
    y2h, y2l = _parts(y2, 2)
    ones_m = jnp.ones((_M, 3), jnp.float32)
    t_aug = jnp.concatenate([t3, ones_m, y2h, y2l], axis=1)

    col_min = jnp.full((1, _M), jnp.inf, dtype=jnp.float32)
    row_total = jnp.float32(0.0)
    for i in range(_N // _NT):
        s3 = src_ref[0, i * _NT:(i + 1) * _NT, :]
        x2 = jnp.sum(s3 * s3, axis=1, keepdims=True)
        x2h, x2m, x2l = _parts(x2, 3)
        s_aug = jnp.concatenate(
            [-2.0 * s3, x2h, x2m, x2l,
             jnp.ones((_NT, 2), jnp.float32)], axis=1)
        d2 = jax.lax.dot_general(
            s_aug, t_aug, (((1,), (1,)), ((), ())),
            precision=jax.lax.Precision.DEFAULT,
            preferred_element_type=jnp.float32)
        row_total = row_total + jnp.sum(jnp.min(d2, axis=1))
        col_min = jnp.minimum(col_min, jnp.min(d2, axis=0, keepdims=True))
    res = row_total / _N + jnp.sum(col_min) / _M

    @pl.when(pl.program_id(0) == 0)
    def _init():
        out_ref[...] = jnp.zeros((1, 1), jnp.float32)

    out_ref[...] = out_ref[...] + jnp.reshape(res, (1, 1))


def kernel(src, tgt):
    total = pl.pallas_call(
        _chamfer_body,
        grid=(_B,),
        in_specs=[
            pl.BlockSpec((1, _N, 3), lambda b: (b, 0, 0)),
            pl.BlockSpec((1, _M, 3), lambda b: (b, 0, 0)),
        ],
        out_specs=pl.BlockSpec((1, 1), lambda b: (0, 0)),
        out_shape=jax.ShapeDtypeStruct((1, 1), jnp.float32),
    )(src, tgt)
    return total[0, 0]


# speedup vs baseline: 2.7544x; 1.0059x over previous
import jax
import jax.numpy as jnp
from jax.experimental import pallas as pl

_B, _N, _M = 16, 2048, 2048
_NT = 1024


def _parts(x, n):
    out = []
    for _ in range(n):
        hi = x.astype(jnp.bfloat16).astype(jnp.float32)
        out.append(hi)
        x = x - hi
    return out


def _chamfer_body(src_ref, tgt_ref, out_ref):
    t3 = tgt_ref[0]
    y2 = jnp.sum(t3 * t3, axis=1, keepdims=True)
    y2h, y2l = _parts(y2, 2)
    ones_m = jnp.ones((_M, 3), jnp.float32)
    t_aug = jnp.concatenate([t3, ones_m, y2h, y2l], axis=1)

    col_min = jnp.full((1, _M), jnp.inf, dtype=jnp.float32)
    row_total = jnp.float32(0.0)
    for i in range(_N // _NT):
        s3 = src_ref[0, i * _NT:(i + 1) * _NT, :]
        x2 = jnp.sum(s3 * s3, axis=1, keepdims=True)
        x2h, x2m, x2l = _parts(x2, 3)
        s_aug = jnp.concatenate(
            [-2.0 * s3, x2h, x2m, x2l,
             jnp.ones((_NT, 2), jnp.float32)], axis=1)
        d2 = jax.lax.dot_general(
            s_aug, t_aug, (((1,), (1,)), ((), ())),
            precision=jax.lax.Precision.DEFAULT,
            preferred_element_type=jnp.float32)
        row_total = row_total + jnp.sum(jnp.min(d2, axis=1))
        col_min = jnp.minimum(col_min, jnp.min(d2, axis=0, keepdims=True))
    res = row_total / _N + jnp.sum(col_min) / _M

    @pl.when(pl.program_id(0) == 0)
    def _init():
        out_ref[...] = jnp.zeros((1, 1), jnp.float32)

    out_ref[...] = out_ref[...] + jnp.reshape(res, (1, 1))


def kernel(src, tgt):
    total = pl.pallas_call(
        _chamfer_body,
        grid=(_B,),
        in_specs=[
            pl.BlockSpec((1, _N, 3), lambda b: (b, 0, 0)),
            pl.BlockSpec((1, _M, 3), lambda b: (b, 0, 0)),
        ],
        out_specs=pl.BlockSpec((1, 1), lambda b: (0, 0)),
        out_shape=jax.ShapeDtypeStruct((1, 1), jnp.float32),
    )(src, tgt)
    return total[0, 0]


# speedup vs baseline: 2.7952x; 1.0148x over previous
import jax
import jax.numpy as jnp
from jax.experimental import pallas as pl

_B, _N, _M = 16, 2048, 2048
_NT = 2048


def _parts(x, n):
    out = []
    for _ in range(n):
        hi = x.astype(jnp.bfloat16).astype(jnp.float32)
        out.append(hi)
        x = x - hi
    return out


def _chamfer_body(src_ref, tgt_ref, out_ref):
    t3 = tgt_ref[0]
    y2 = jnp.sum(t3 * t3, axis=1, keepdims=True)
    y2h, y2l = _parts(y2, 2)
    ones_m = jnp.ones((_M, 3), jnp.float32)
    t_aug = jnp.concatenate([t3, ones_m, y2h, y2l], axis=1)

    col_min = jnp.full((1, _M), jnp.inf, dtype=jnp.float32)
    row_total = jnp.float32(0.0)
    for i in range(_N // _NT):
        s3 = src_ref[0, i * _NT:(i + 1) * _NT, :]
        x2 = jnp.sum(s3 * s3, axis=1, keepdims=True)
        x2h, x2m, x2l = _parts(x2, 3)
        s_aug = jnp.concatenate(
            [-2.0 * s3, x2h, x2m, x2l,
             jnp.ones((_NT, 2), jnp.float32)], axis=1)
        d2 = jax.lax.dot_general(
            s_aug, t_aug, (((1,), (1,)), ((), ())),
            precision=jax.lax.Precision.DEFAULT,
            preferred_element_type=jnp.float32)
        row_total = row_total + jnp.sum(jnp.min(d2, axis=1))
        col_min = jnp.minimum(col_min, jnp.min(d2, axis=0, keepdims=True))
    res = row_total / _N + jnp.sum(col_min) / _M

    @pl.when(pl.program_id(0) == 0)
    def _init():
        out_ref[...] = jnp.zeros((1, 1), jnp.float32)

    out_ref[...] = out_ref[...] + jnp.reshape(res, (1, 1))


def kernel(src, tgt):
    total = pl.pallas_call(
        _chamfer_body,
        grid=(_B,),
        in_specs=[
            pl.BlockSpec((1, _N, 3), lambda b: (b, 0, 0)),
            pl.BlockSpec((1, _M, 3), lambda b: (b, 0, 0)),
        ],
        out_specs=pl.BlockSpec((1, 1), lambda b: (0, 0)),
        out_shape=jax.ShapeDtypeStruct((1, 1), jnp.float32),
    )(src, tgt)
    return total[0, 0]


# speedup vs baseline: 3.9603x; 1.4168x over previous
import jax
import jax.numpy as jnp
from jax.experimental import pallas as pl

_B, _N, _M = 16, 2048, 2048


def _parts(x, n):
    out = []
    for _ in range(n):
        hi = x.astype(jnp.bfloat16).astype(jnp.float32)
        out.append(hi)
        x = x - hi
    return out


def _aug(c3, extra_rows):
    return jnp.concatenate([c3] + extra_rows, axis=0)


def _chamfer_body(srcT_ref, tgtT_ref, out_ref):
    s3 = srcT_ref[0]
    t3 = tgtT_ref[0]
    x2 = jnp.sum(s3 * s3, axis=0, keepdims=True)
    y2 = jnp.sum(t3 * t3, axis=0, keepdims=True)
    x2h, x2m, x2l = _parts(x2, 3)
    y2h, y2l = _parts(y2, 2)
    s_aug = _aug(-2.0 * s3,
                 [x2h, x2m, x2l, jnp.ones((2, _N), jnp.float32)])
    t_aug = _aug(t3,
                 [jnp.ones((3, _M), jnp.float32), y2h, y2l])
    d2 = jax.lax.dot_general(
        s_aug, t_aug, (((0,), (0,)), ((), ())),
        precision=jax.lax.Precision.DEFAULT,
        preferred_element_type=jnp.float32)
    row_total = jnp.sum(jnp.min(d2, axis=1))
    col_total = jnp.sum(jnp.min(d2, axis=0))
    res = row_total / _N + col_total / _M

    @pl.when(pl.program_id(0) == 0)
    def _init():
        out_ref[...] = jnp.zeros((1, 1), jnp.float32)

    out_ref[...] = out_ref[...] + jnp.reshape(res, (1, 1))


def kernel(src, tgt):
    srcT = jnp.transpose(src, (0, 2, 1))
    tgtT = jnp.transpose(tgt, (0, 2, 1))
    total = pl.pallas_call(
        _chamfer_body,
        grid=(_B,),
        in_specs=[
            pl.BlockSpec((1, 3, _N), lambda b: (b, 0, 0)),
            pl.BlockSpec((1, 3, _M), lambda b: (b, 0, 0)),
        ],
        out_specs=pl.BlockSpec((1, 1), lambda b: (0, 0)),
        out_shape=jax.ShapeDtypeStruct((1, 1), jnp.float32),
    )(srcT, tgtT)
    return total[0, 0]
